# baseline (speedup 1.0000x reference)
import functools

import jax
import jax.numpy as jnp
from jax import lax
from jax.experimental import pallas as pl
from jax.experimental.pallas import tpu as pltpu
from jax.experimental.pallas import tpu_sc as plsc

G = 2
S = 2048
D = 1024
H = 2048
E = 8
CAP = 512
NSLOT = E * CAP
ZROWS = 256
XPAD = S + ZROWS
NWORKERS = 32
LANES = 16
SLOT_BLK = 512


def _router_body(x_ref, w_ref, b_ref, gates_ref, destc_ref, inv_ref):
    g = pl.program_id(0)
    xg = x_ref[0]
    w = w_ref[...]
    b = b_ref[...]
    lt = lax.dot_general(w, xg, (((0,), (1,)), ((), ())),
                         preferred_element_type=jnp.float32)
    lt = lt + b

    eidx = lax.broadcasted_iota(jnp.int32, (E, S), 0)
    m1 = jnp.max(lt, axis=0, keepdims=True)
    am1 = jnp.min(jnp.where(lt == m1, eidx, E), axis=0, keepdims=True)
    lt2 = jnp.where(eidx == am1, jnp.float32(-1e30), lt)
    m2 = jnp.max(lt2, axis=0, keepdims=True)
    am2 = jnp.min(jnp.where(lt2 == m2, eidx, E), axis=0, keepdims=True)

    ssum = jnp.sum(jnp.exp(lt - m1), axis=0, keepdims=True)
    gate1 = 1.0 / ssum
    gate2 = jnp.exp(m2 - m1) / ssum

    oh1 = (eidx == am1).astype(jnp.float32)
    oh2 = (eidx == am2).astype(jnp.float32)
    m16 = jnp.concatenate([oh1, oh2], axis=0)

    ch = 512
    ut = (lax.broadcasted_iota(jnp.int32, (ch, ch), 0)
          <= lax.broadcasted_iota(jnp.int32, (ch, ch), 1)).astype(jnp.float32)
    prefix = jnp.zeros((16, 1), jnp.float32)
    chunks = []
    for i in range(S // ch):
        seg = m16[:, i * ch:(i + 1) * ch]
        cs = lax.dot_general(seg, ut, (((1,), (0,)), ((), ())),
                             preferred_element_type=jnp.float32) + prefix
        chunks.append(cs)
        prefix = cs[:, ch - 1:ch]
    csum = jnp.concatenate(chunks, axis=1)

    pos1 = jnp.sum(oh1 * csum[0:E], axis=0, keepdims=True)
    pos2 = jnp.sum(oh2 * csum[E:2 * E], axis=0, keepdims=True)

    valid1 = pos1 < jnp.float32(CAP)
    valid2 = pos2 < jnp.float32(CAP)
    gv1 = gate1 * valid1.astype(jnp.float32)
    gv2 = gate2 * valid2.astype(jnp.float32)
    d1 = jnp.where(valid1, am1 * CAP + pos1.astype(jnp.int32) - 1, NSLOT)
    d2 = jnp.where(valid2, am2 * CAP + pos2.astype(jnp.int32) - 1, NSLOT)

    gates_ref[0] = jnp.concatenate([gv1, gv2], axis=0)
    destc_ref[0] = jnp.concatenate(
        [jnp.minimum(d1, NSLOT - 1), jnp.minimum(d2, NSLOT - 1)],
        axis=0) + g * NSLOT

    zrow = g * XPAD + S
    svrow = (lax.broadcasted_iota(jnp.int32, (1, S), 1)
             + (g * XPAD - zrow))
    for k, drow in ((0, d1), (1, d2)):
        for blk in range(NSLOT // SLOT_BLK):
            siota = (lax.broadcasted_iota(jnp.int32, (SLOT_BLK, 1), 0)
                     + blk * SLOT_BLK)
            mb = (drow == siota)
            invcol = jnp.sum(jnp.where(mb, svrow, 0), axis=1, keepdims=True)
            spread = jnp.where(invcol == 0, siota % ZROWS, 0)
            inv_ref[0, k, blk] = invcol + spread + zrow


def _router(x, gate_weight, gate_bias2):
    nblk = NSLOT // SLOT_BLK
    return pl.pallas_call(
        _router_body,
        grid=(G,),
        in_specs=[
            pl.BlockSpec((1, S, D), lambda g: (g, 0, 0)),
            pl.BlockSpec((D, E), lambda g: (0, 0)),
            pl.BlockSpec((E, 1), lambda g: (0, 0)),
        ],
        out_specs=[
            pl.BlockSpec((1, 2, S), lambda g: (g, 0, 0)),
            pl.BlockSpec((1, 2, S), lambda g: (g, 0, 0)),
            pl.BlockSpec((1, 2, nblk, SLOT_BLK, 1), lambda g: (g, 0, 0, 0, 0)),
        ],
        out_shape=[
            jax.ShapeDtypeStruct((G, 2, S), jnp.float32),
            jax.ShapeDtypeStruct((G, 2, S), jnp.int32),
            jax.ShapeDtypeStruct((G, 2, nblk, SLOT_BLK, 1), jnp.int32),
        ],
    )(x, gate_weight, gate_bias2)


_SLOTS_PER_W = G * NSLOT // NWORKERS
_DCHUNK = 16


def _dispatch_body(xpad, inv_hbm, a0, a1,
                   idx0v, idx1v, rows0, rows1, sem0, sem1):
    wid = lax.axis_index("s") * 2 + lax.axis_index("c")
    g = wid // (NWORKERS // G)
    lbase = (wid % (NWORKERS // G)) * _SLOTS_PER_W

    pltpu.sync_copy(inv_hbm.at[g, 0, pl.ds(lbase, _SLOTS_PER_W)], idx0v)
    pltpu.sync_copy(inv_hbm.at[g, 1, pl.ds(lbase, _SLOTS_PER_W)], idx1v)

    def gbody(j, _):
        off = j * _DCHUNK
        i0 = idx0v[pl.ds(off, _DCHUNK)]
        i1 = idx1v[pl.ds(off, _DCHUNK)]
        cp0 = pltpu.async_copy(xpad.at[i0], rows0, sem0)
        cp1 = pltpu.async_copy(xpad.at[i1], rows1, sem1)
        cp0.wait()
        cp1.wait()
        grow = g * NSLOT + lbase + off
        pltpu.sync_copy(rows0, a0.at[pl.ds(grow, _DCHUNK)])
        pltpu.sync_copy(rows1, a1.at[pl.ds(grow, _DCHUNK)])
        return 0
    lax.fori_loop(0, _SLOTS_PER_W // _DCHUNK, gbody, 0)


def _dispatch(xpad, inv):
    mesh = plsc.VectorSubcoreMesh(core_axis_name="c", subcore_axis_name="s")
    f = functools.partial(
        pl.kernel,
        out_type=[jax.ShapeDtypeStruct((G * NSLOT, D), jnp.float32),
                  jax.ShapeDtypeStruct((G * NSLOT, D), jnp.float32)],
        mesh=mesh,
        scratch_types=[
            pltpu.VMEM((_SLOTS_PER_W,), jnp.int32),
            pltpu.VMEM((_SLOTS_PER_W,), jnp.int32),
            pltpu.VMEM((_DCHUNK, D), jnp.float32),
            pltpu.VMEM((_DCHUNK, D), jnp.float32),
            pltpu.SemaphoreType.DMA,
            pltpu.SemaphoreType.DMA,
        ],
    )(_dispatch_body)
    return f(xpad, inv)


def _ffn_body(a0_ref, a1_ref, k_ref, v_ref, o_ref):
    a = (a0_ref[0, 0] + a1_ref[0, 0]).astype(jnp.bfloat16)
    h = jnp.dot(a, k_ref[0].astype(jnp.bfloat16),
                preferred_element_type=jnp.float32)
    h = jax.nn.gelu(h).astype(jnp.bfloat16)
    o_ref[0, 0] = jnp.dot(h, v_ref[0].astype(jnp.bfloat16),
                          preferred_element_type=jnp.float32)


def _ffn(a0, a1, expert_keys, expert_values):
    return pl.pallas_call(
        _ffn_body,
        grid=(E, G),
        in_specs=[
            pl.BlockSpec((1, 1, CAP, D), lambda e, g: (g, e, 0, 0)),
            pl.BlockSpec((1, 1, CAP, D), lambda e, g: (g, e, 0, 0)),
            pl.BlockSpec((1, D, H), lambda e, g: (e, 0, 0)),
            pl.BlockSpec((1, H, D), lambda e, g: (e, 0, 0)),
        ],
        out_specs=pl.BlockSpec((1, 1, CAP, D), lambda e, g: (g, e, 0, 0)),
        out_shape=jax.ShapeDtypeStruct((G, E, CAP, D), jnp.float32),
    )(a0, a1, expert_keys, expert_values)


_TOK_PER_W = G * S // NWORKERS
_CCHUNK = 16


def _cgather_body(aout, destc_hbm, r0, r1,
                  d0v, d1v, rows0, rows1, sem0, sem1):
    wid = lax.axis_index("s") * 2 + lax.axis_index("c")
    g = wid // (NWORKERS // G)
    tbase = (wid % (NWORKERS // G)) * _TOK_PER_W

    pltpu.sync_copy(destc_hbm.at[g, 0, pl.ds(tbase, _TOK_PER_W)], d0v)
    pltpu.sync_copy(destc_hbm.at[g, 1, pl.ds(tbase, _TOK_PER_W)], d1v)

    def body(j, _):
        off = j * _CCHUNK
        i0 = d0v[pl.ds(off, _CCHUNK)]
        i1 = d1v[pl.ds(off, _CCHUNK)]
        cp0 = pltpu.async_copy(aout.at[i0], rows0, sem0)
        cp1 = pltpu.async_copy(aout.at[i1], rows1, sem1)
        cp0.wait()
        cp1.wait()
        trow = g * S + tbase + off
        pltpu.sync_copy(rows0, r0.at[pl.ds(trow, _CCHUNK)])
        pltpu.sync_copy(rows1, r1.at[pl.ds(trow, _CCHUNK)])
        return 0
    lax.fori_loop(0, _TOK_PER_W // _CCHUNK, body, 0)


def _cgather(aout_flat, destc):
    mesh = plsc.VectorSubcoreMesh(core_axis_name="c", subcore_axis_name="s")
    f = functools.partial(
        pl.kernel,
        out_type=[jax.ShapeDtypeStruct((G * S, D), jnp.float32),
                  jax.ShapeDtypeStruct((G * S, D), jnp.float32)],
        mesh=mesh,
        scratch_types=[
            pltpu.VMEM((_TOK_PER_W,), jnp.int32),
            pltpu.VMEM((_TOK_PER_W,), jnp.int32),
            pltpu.VMEM((_CCHUNK, D), jnp.float32),
            pltpu.VMEM((_CCHUNK, D), jnp.float32),
            pltpu.SemaphoreType.DMA,
            pltpu.SemaphoreType.DMA,
        ],
    )(_cgather_body)
    return f(aout_flat, destc)


_SB = 512


def _cscale_body(r0_ref, r1_ref, g_ref, o_ref):
    gv = g_ref[0]
    o_ref[0] = gv[:, 0:1] * r0_ref[0] + gv[:, 1:2] * r1_ref[0]


def _cscale(r0, r1, gates_c):
    return pl.pallas_call(
        _cscale_body,
        grid=(G, S // _SB),
        in_specs=[
            pl.BlockSpec((1, _SB, D), lambda g, s: (g, s, 0)),
            pl.BlockSpec((1, _SB, D), lambda g, s: (g, s, 0)),
            pl.BlockSpec((1, _SB, 2), lambda g, s: (g, s, 0)),
        ],
        out_specs=pl.BlockSpec((1, _SB, D), lambda g, s: (g, s, 0)),
        out_shape=jax.ShapeDtypeStruct((G, S, D), jnp.float32),
    )(r0, r1, gates_c)


def kernel(x, gate_weight, gate_bias, expert_keys, expert_values):
    xpad = jnp.concatenate(
        [x, jnp.zeros((G, XPAD - S, D), x.dtype)], axis=1).reshape(G * XPAD, D)
    gates, destc, inv5 = _router(x, gate_weight, gate_bias.reshape(E, 1))
    inv = inv5.reshape(G, 2, NSLOT)
    a0, a1 = _dispatch(xpad, inv)
    aout = _ffn(a0.reshape(G, E, CAP, D), a1.reshape(G, E, CAP, D),
                expert_keys, expert_values)
    r0, r1 = _cgather(aout.reshape(G * NSLOT, D), destc)
    out = _cscale(r0.reshape(G, S, D), r1.reshape(G, S, D),
                  jnp.transpose(gates, (0, 2, 1)))
    return out


# speedup vs baseline: 1.0413x; 1.0413x over previous
import functools

import jax
import jax.numpy as jnp
from jax import lax
from jax.experimental import pallas as pl
from jax.experimental.pallas import tpu as pltpu
from jax.experimental.pallas import tpu_sc as plsc

G = 2
S = 2048
D = 1024
H = 2048
E = 8
CAP = 512
NSLOT = E * CAP
ZROWS = 256
XPAD = S + ZROWS
NWORKERS = 32
LANES = 16
SLOT_BLK = 512


def _router_body(x_ref, w_ref, b_ref, gates_ref, destc_ref, inv_ref,
                 xpad_ref):
    g = pl.program_id(0)
    xg = x_ref[0]
    xpad_ref[0, :S] = xg
    xpad_ref[0, S:] = jnp.zeros((XPAD - S, D), jnp.float32)
    w = w_ref[...]
    b = b_ref[...]
    lt = lax.dot_general(w, xg, (((0,), (1,)), ((), ())),
                         preferred_element_type=jnp.float32)
    lt = lt + b

    eidx = lax.broadcasted_iota(jnp.int32, (E, S), 0)
    m1 = jnp.max(lt, axis=0, keepdims=True)
    am1 = jnp.min(jnp.where(lt == m1, eidx, E), axis=0, keepdims=True)
    lt2 = jnp.where(eidx == am1, jnp.float32(-1e30), lt)
    m2 = jnp.max(lt2, axis=0, keepdims=True)
    am2 = jnp.min(jnp.where(lt2 == m2, eidx, E), axis=0, keepdims=True)

    ssum = jnp.sum(jnp.exp(lt - m1), axis=0, keepdims=True)
    gate1 = 1.0 / ssum
    gate2 = jnp.exp(m2 - m1) / ssum

    oh1 = (eidx == am1).astype(jnp.float32)
    oh2 = (eidx == am2).astype(jnp.float32)
    m16 = jnp.concatenate([oh1, oh2], axis=0)

    ch = 512
    ut = (lax.broadcasted_iota(jnp.int32, (ch, ch), 0)
          <= lax.broadcasted_iota(jnp.int32, (ch, ch), 1)).astype(jnp.float32)
    prefix = jnp.zeros((16, 1), jnp.float32)
    chunks = []
    for i in range(S // ch):
        seg = m16[:, i * ch:(i + 1) * ch]
        cs = lax.dot_general(seg, ut, (((1,), (0,)), ((), ())),
                             preferred_element_type=jnp.float32) + prefix
        chunks.append(cs)
        prefix = cs[:, ch - 1:ch]
    csum = jnp.concatenate(chunks, axis=1)

    pos1 = jnp.sum(oh1 * csum[0:E], axis=0, keepdims=True)
    pos2 = jnp.sum(oh2 * csum[E:2 * E], axis=0, keepdims=True)

    valid1 = pos1 < jnp.float32(CAP)
    valid2 = pos2 < jnp.float32(CAP)
    gv1 = gate1 * valid1.astype(jnp.float32)
    gv2 = gate2 * valid2.astype(jnp.float32)
    d1 = jnp.where(valid1, am1 * CAP + pos1.astype(jnp.int32) - 1, NSLOT)
    d2 = jnp.where(valid2, am2 * CAP + pos2.astype(jnp.int32) - 1, NSLOT)

    gates_ref[0] = jnp.concatenate([gv1, gv2], axis=0)
    destc_ref[0] = jnp.concatenate(
        [jnp.minimum(d1, NSLOT - 1), jnp.minimum(d2, NSLOT - 1)],
        axis=0) + g * NSLOT

    zrow = g * XPAD + S
    svrow = (lax.broadcasted_iota(jnp.int32, (1, S), 1)
             + (g * XPAD - zrow))
    for k, drow in ((0, d1), (1, d2)):
        for blk in range(NSLOT // SLOT_BLK):
            siota = (lax.broadcasted_iota(jnp.int32, (SLOT_BLK, 1), 0)
                     + blk * SLOT_BLK)
            mb = (drow == siota)
            invcol = jnp.sum(jnp.where(mb, svrow, 0), axis=1, keepdims=True)
            spread = jnp.where(invcol == 0, siota % ZROWS, 0)
            inv_ref[0, k, blk] = invcol + spread + zrow


def _router(x, gate_weight, gate_bias2):
    nblk = NSLOT // SLOT_BLK
    return pl.pallas_call(
        _router_body,
        grid=(G,),
        in_specs=[
            pl.BlockSpec((1, S, D), lambda g: (g, 0, 0)),
            pl.BlockSpec((D, E), lambda g: (0, 0)),
            pl.BlockSpec((E, 1), lambda g: (0, 0)),
        ],
        out_specs=[
            pl.BlockSpec((1, 2, S), lambda g: (g, 0, 0)),
            pl.BlockSpec((1, 2, S), lambda g: (g, 0, 0)),
            pl.BlockSpec((1, 2, nblk, SLOT_BLK, 1), lambda g: (g, 0, 0, 0, 0)),
            pl.BlockSpec((1, XPAD, D), lambda g: (g, 0, 0)),
        ],
        out_shape=[
            jax.ShapeDtypeStruct((G, 2, S), jnp.float32),
            jax.ShapeDtypeStruct((G, 2, S), jnp.int32),
            jax.ShapeDtypeStruct((G, 2, nblk, SLOT_BLK, 1), jnp.int32),
            jax.ShapeDtypeStruct((G, XPAD, D), jnp.float32),
        ],
    )(x, gate_weight, gate_bias2)


_SLOTS_PER_W = G * NSLOT // NWORKERS
_DCHUNK = 16


def _dispatch_body(xpad, inv_hbm, a0, a1,
                   idx0v, idx1v, rows0, rows1, sem0, sem1):
    wid = lax.axis_index("s") * 2 + lax.axis_index("c")
    g = wid // (NWORKERS // G)
    lbase = (wid % (NWORKERS // G)) * _SLOTS_PER_W

    pltpu.sync_copy(inv_hbm.at[g, 0, pl.ds(lbase, _SLOTS_PER_W)], idx0v)
    pltpu.sync_copy(inv_hbm.at[g, 1, pl.ds(lbase, _SLOTS_PER_W)], idx1v)

    def gbody(j, _):
        off = j * _DCHUNK
        i0 = idx0v[pl.ds(off, _DCHUNK)]
        i1 = idx1v[pl.ds(off, _DCHUNK)]
        cp0 = pltpu.async_copy(xpad.at[i0], rows0, sem0)
        cp1 = pltpu.async_copy(xpad.at[i1], rows1, sem1)
        cp0.wait()
        cp1.wait()
        grow = g * NSLOT + lbase + off
        pltpu.sync_copy(rows0, a0.at[pl.ds(grow, _DCHUNK)])
        pltpu.sync_copy(rows1, a1.at[pl.ds(grow, _DCHUNK)])
        return 0
    lax.fori_loop(0, _SLOTS_PER_W // _DCHUNK, gbody, 0)


def _dispatch(xpad, inv):
    mesh = plsc.VectorSubcoreMesh(core_axis_name="c", subcore_axis_name="s")
    f = functools.partial(
        pl.kernel,
        out_type=[jax.ShapeDtypeStruct((G * NSLOT, D), jnp.float32),
                  jax.ShapeDtypeStruct((G * NSLOT, D), jnp.float32)],
        mesh=mesh,
        scratch_types=[
            pltpu.VMEM((_SLOTS_PER_W,), jnp.int32),
            pltpu.VMEM((_SLOTS_PER_W,), jnp.int32),
            pltpu.VMEM((_DCHUNK, D), jnp.float32),
            pltpu.VMEM((_DCHUNK, D), jnp.float32),
            pltpu.SemaphoreType.DMA,
            pltpu.SemaphoreType.DMA,
        ],
    )(_dispatch_body)
    return f(xpad, inv)


def _ffn_body(a0_ref, a1_ref, k_ref, v_ref, o_ref):
    a = (a0_ref[0, 0] + a1_ref[0, 0]).astype(jnp.bfloat16)
    h = jnp.dot(a, k_ref[0].astype(jnp.bfloat16),
                preferred_element_type=jnp.float32)
    h = jax.nn.gelu(h).astype(jnp.bfloat16)
    o_ref[0, 0] = jnp.dot(h, v_ref[0].astype(jnp.bfloat16),
                          preferred_element_type=jnp.float32)


def _ffn(a0, a1, expert_keys, expert_values):
    return pl.pallas_call(
        _ffn_body,
        grid=(E, G),
        in_specs=[
            pl.BlockSpec((1, 1, CAP, D), lambda e, g: (g, e, 0, 0)),
            pl.BlockSpec((1, 1, CAP, D), lambda e, g: (g, e, 0, 0)),
            pl.BlockSpec((1, D, H), lambda e, g: (e, 0, 0)),
            pl.BlockSpec((1, H, D), lambda e, g: (e, 0, 0)),
        ],
        out_specs=pl.BlockSpec((1, 1, CAP, D), lambda e, g: (g, e, 0, 0)),
        out_shape=jax.ShapeDtypeStruct((G, E, CAP, D), jnp.float32),
    )(a0, a1, expert_keys, expert_values)


_TOK_PER_W = G * S // NWORKERS
_CCHUNK = 16


def _cgather_body(aout, destc_hbm, r0, r1,
                  d0v, d1v, rows0, rows1, sem0, sem1):
    wid = lax.axis_index("s") * 2 + lax.axis_index("c")
    g = wid // (NWORKERS // G)
    tbase = (wid % (NWORKERS // G)) * _TOK_PER_W

    pltpu.sync_copy(destc_hbm.at[g, 0, pl.ds(tbase, _TOK_PER_W)], d0v)
    pltpu.sync_copy(destc_hbm.at[g, 1, pl.ds(tbase, _TOK_PER_W)], d1v)

    def body(j, _):
        off = j * _CCHUNK
        i0 = d0v[pl.ds(off, _CCHUNK)]
        i1 = d1v[pl.ds(off, _CCHUNK)]
        cp0 = pltpu.async_copy(aout.at[i0], rows0, sem0)
        cp1 = pltpu.async_copy(aout.at[i1], rows1, sem1)
        cp0.wait()
        cp1.wait()
        trow = g * S + tbase + off
        pltpu.sync_copy(rows0, r0.at[pl.ds(trow, _CCHUNK)])
        pltpu.sync_copy(rows1, r1.at[pl.ds(trow, _CCHUNK)])
        return 0
    lax.fori_loop(0, _TOK_PER_W // _CCHUNK, body, 0)


def _cgather(aout_flat, destc):
    mesh = plsc.VectorSubcoreMesh(core_axis_name="c", subcore_axis_name="s")
    f = functools.partial(
        pl.kernel,
        out_type=[jax.ShapeDtypeStruct((G * S, D), jnp.float32),
                  jax.ShapeDtypeStruct((G * S, D), jnp.float32)],
        mesh=mesh,
        scratch_types=[
            pltpu.VMEM((_TOK_PER_W,), jnp.int32),
            pltpu.VMEM((_TOK_PER_W,), jnp.int32),
            pltpu.VMEM((_CCHUNK, D), jnp.float32),
            pltpu.VMEM((_CCHUNK, D), jnp.float32),
            pltpu.SemaphoreType.DMA,
            pltpu.SemaphoreType.DMA,
        ],
    )(_cgather_body)
    return f(aout_flat, destc)


_SB = 512


def _cscale_body(r0_ref, r1_ref, g_ref, o_ref):
    gv = g_ref[0]
    o_ref[0] = gv[:, 0:1] * r0_ref[0] + gv[:, 1:2] * r1_ref[0]


def _cscale(r0, r1, gates_c):
    return pl.pallas_call(
        _cscale_body,
        grid=(G, S // _SB),
        in_specs=[
            pl.BlockSpec((1, _SB, D), lambda g, s: (g, s, 0)),
            pl.BlockSpec((1, _SB, D), lambda g, s: (g, s, 0)),
            pl.BlockSpec((1, _SB, 2), lambda g, s: (g, s, 0)),
        ],
        out_specs=pl.BlockSpec((1, _SB, D), lambda g, s: (g, s, 0)),
        out_shape=jax.ShapeDtypeStruct((G, S, D), jnp.float32),
    )(r0, r1, gates_c)


def kernel(x, gate_weight, gate_bias, expert_keys, expert_values):
    gates, destc, inv5, xpad3 = _router(x, gate_weight, gate_bias.reshape(E, 1))
    xpad = xpad3.reshape(G * XPAD, D)
    inv = inv5.reshape(G, 2, NSLOT)
    a0, a1 = _dispatch(xpad, inv)
    aout = _ffn(a0.reshape(G, E, CAP, D), a1.reshape(G, E, CAP, D),
                expert_keys, expert_values)
    r0, r1 = _cgather(aout.reshape(G * NSLOT, D), destc)
    out = _cscale(r0.reshape(G, S, D), r1.reshape(G, S, D),
                  jnp.transpose(gates, (0, 2, 1)))
    return out


# speedup vs baseline: 1.0944x; 1.0511x over previous
import functools

import jax
import jax.numpy as jnp
from jax import lax
from jax.experimental import pallas as pl
from jax.experimental.pallas import tpu as pltpu
from jax.experimental.pallas import tpu_sc as plsc

G = 2
S = 2048
D = 1024
H = 2048
E = 8
CAP = 512
NSLOT = E * CAP
ZROWS = 256
XPAD = S + ZROWS
NWORKERS = 32
LANES = 16
SLOT_BLK = 512


def _router_body(x_ref, w_ref, b_ref, gates_ref, destc_ref, inv_ref,
                 xpad_ref):
    g = pl.program_id(0)
    xg = x_ref[0]
    xpad_ref[0, :S] = xg
    xpad_ref[0, S:] = jnp.zeros((XPAD - S, D), jnp.float32)
    w = w_ref[...]
    b = b_ref[...]
    lt = lax.dot_general(w, xg, (((0,), (1,)), ((), ())),
                         preferred_element_type=jnp.float32)
    lt = lt + b

    eidx = lax.broadcasted_iota(jnp.int32, (E, S), 0)
    m1 = jnp.max(lt, axis=0, keepdims=True)
    am1 = jnp.min(jnp.where(lt == m1, eidx, E), axis=0, keepdims=True)
    lt2 = jnp.where(eidx == am1, jnp.float32(-1e30), lt)
    m2 = jnp.max(lt2, axis=0, keepdims=True)
    am2 = jnp.min(jnp.where(lt2 == m2, eidx, E), axis=0, keepdims=True)

    ssum = jnp.sum(jnp.exp(lt - m1), axis=0, keepdims=True)
    gate1 = 1.0 / ssum
    gate2 = jnp.exp(m2 - m1) / ssum

    oh1 = (eidx == am1).astype(jnp.float32)
    oh2 = (eidx == am2).astype(jnp.float32)
    m16 = jnp.concatenate([oh1, oh2], axis=0)

    ch = 512
    ut = (lax.broadcasted_iota(jnp.int32, (ch, ch), 0)
          <= lax.broadcasted_iota(jnp.int32, (ch, ch), 1)).astype(jnp.float32)
    prefix = jnp.zeros((16, 1), jnp.float32)
    chunks = []
    for i in range(S // ch):
        seg = m16[:, i * ch:(i + 1) * ch]
        cs = lax.dot_general(seg, ut, (((1,), (0,)), ((), ())),
                             preferred_element_type=jnp.float32) + prefix
        chunks.append(cs)
        prefix = cs[:, ch - 1:ch]
    csum = jnp.concatenate(chunks, axis=1)

    pos1 = jnp.sum(oh1 * csum[0:E], axis=0, keepdims=True)
    pos2 = jnp.sum(oh2 * csum[E:2 * E], axis=0, keepdims=True)

    valid1 = pos1 < jnp.float32(CAP)
    valid2 = pos2 < jnp.float32(CAP)
    gv1 = gate1 * valid1.astype(jnp.float32)
    gv2 = gate2 * valid2.astype(jnp.float32)
    d1 = jnp.where(valid1, am1 * CAP + pos1.astype(jnp.int32) - 1, NSLOT)
    d2 = jnp.where(valid2, am2 * CAP + pos2.astype(jnp.int32) - 1, NSLOT)

    gates_ref[0] = jnp.concatenate([gv1, gv2], axis=0)
    destc_ref[0] = jnp.concatenate(
        [jnp.minimum(d1, NSLOT - 1), jnp.minimum(d2, NSLOT - 1)],
        axis=0) + g * NSLOT

    zrow = g * XPAD + S
    svrow = (lax.broadcasted_iota(jnp.int32, (1, S), 1)
             + (g * XPAD - zrow))
    for k, drow in ((0, d1), (1, d2)):
        for blk in range(NSLOT // SLOT_BLK):
            siota = (lax.broadcasted_iota(jnp.int32, (SLOT_BLK, 1), 0)
                     + blk * SLOT_BLK)
            mb = (drow == siota)
            invcol = jnp.sum(jnp.where(mb, svrow, 0), axis=1, keepdims=True)
            spread = jnp.where(invcol == 0, siota % ZROWS, 0)
            inv_ref[0, k, blk] = invcol + spread + zrow


def _router(x, gate_weight, gate_bias2):
    nblk = NSLOT // SLOT_BLK
    return pl.pallas_call(
        _router_body,
        grid=(G,),
        in_specs=[
            pl.BlockSpec((1, S, D), lambda g: (g, 0, 0)),
            pl.BlockSpec((D, E), lambda g: (0, 0)),
            pl.BlockSpec((E, 1), lambda g: (0, 0)),
        ],
        out_specs=[
            pl.BlockSpec((1, 2, S), lambda g: (g, 0, 0)),
            pl.BlockSpec((1, 2, S), lambda g: (g, 0, 0)),
            pl.BlockSpec((1, 2, nblk, SLOT_BLK, 1), lambda g: (g, 0, 0, 0, 0)),
            pl.BlockSpec((1, XPAD, D), lambda g: (g, 0, 0)),
        ],
        out_shape=[
            jax.ShapeDtypeStruct((G, 2, S), jnp.float32),
            jax.ShapeDtypeStruct((G, 2, S), jnp.int32),
            jax.ShapeDtypeStruct((G, 2, nblk, SLOT_BLK, 1), jnp.int32),
            jax.ShapeDtypeStruct((G, XPAD, D), jnp.float32),
        ],
    )(x, gate_weight, gate_bias2)


HNS = NSLOT // 2
_SLOTS_PER_W = G * HNS // NWORKERS
_DCHUNK = 16


def _dispatch_half_body(soff, xpad, inv_hbm, a0, a1,
                        idx0v, idx1v, rows0, rows1, sem0, sem1):
    wid = lax.axis_index("s") * 2 + lax.axis_index("c")
    g = wid // (NWORKERS // G)
    lbase = (wid % (NWORKERS // G)) * _SLOTS_PER_W

    pltpu.sync_copy(inv_hbm.at[g, 0, pl.ds(soff + lbase, _SLOTS_PER_W)], idx0v)
    pltpu.sync_copy(inv_hbm.at[g, 1, pl.ds(soff + lbase, _SLOTS_PER_W)], idx1v)

    def gbody(j, _):
        off = j * _DCHUNK
        i0 = idx0v[pl.ds(off, _DCHUNK)]
        i1 = idx1v[pl.ds(off, _DCHUNK)]
        cp0 = pltpu.async_copy(xpad.at[i0], rows0, sem0)
        cp1 = pltpu.async_copy(xpad.at[i1], rows1, sem1)
        cp0.wait()
        cp1.wait()
        grow = g * HNS + lbase + off
        pltpu.sync_copy(rows0, a0.at[pl.ds(grow, _DCHUNK)])
        pltpu.sync_copy(rows1, a1.at[pl.ds(grow, _DCHUNK)])
        return 0
    lax.fori_loop(0, _SLOTS_PER_W // _DCHUNK, gbody, 0)


def _dispatch_half(xpad, inv, soff):
    mesh = plsc.VectorSubcoreMesh(core_axis_name="c", subcore_axis_name="s")
    f = functools.partial(
        pl.kernel,
        out_type=[jax.ShapeDtypeStruct((G * HNS, D), jnp.float32),
                  jax.ShapeDtypeStruct((G * HNS, D), jnp.float32)],
        mesh=mesh,
        scratch_types=[
            pltpu.VMEM((_SLOTS_PER_W,), jnp.int32),
            pltpu.VMEM((_SLOTS_PER_W,), jnp.int32),
            pltpu.VMEM((_DCHUNK, D), jnp.float32),
            pltpu.VMEM((_DCHUNK, D), jnp.float32),
            pltpu.SemaphoreType.DMA,
            pltpu.SemaphoreType.DMA,
        ],
    )(functools.partial(_dispatch_half_body, soff))
    return f(xpad, inv)


def _ffn_body(a0_ref, a1_ref, k_ref, v_ref, o_ref):
    a = (a0_ref[0, 0] + a1_ref[0, 0]).astype(jnp.bfloat16)
    h = jnp.dot(a, k_ref[0].astype(jnp.bfloat16),
                preferred_element_type=jnp.float32)
    h = jax.nn.gelu(h).astype(jnp.bfloat16)
    o_ref[0, 0] = jnp.dot(h, v_ref[0].astype(jnp.bfloat16),
                          preferred_element_type=jnp.float32)


def _ffn_b_body(a0_ref, a1_ref, k_ref, v_ref, prev_ref, o_ref):
    del prev_ref
    _ffn_body(a0_ref, a1_ref, k_ref, v_ref, o_ref)


def _ffn_half_a(a0, a1, expert_keys, expert_values):
    return pl.pallas_call(
        _ffn_body,
        grid=(E // 2, G),
        in_specs=[
            pl.BlockSpec((1, 1, CAP, D), lambda e, g: (g, e, 0, 0)),
            pl.BlockSpec((1, 1, CAP, D), lambda e, g: (g, e, 0, 0)),
            pl.BlockSpec((1, D, H), lambda e, g: (e, 0, 0)),
            pl.BlockSpec((1, H, D), lambda e, g: (e, 0, 0)),
        ],
        out_specs=pl.BlockSpec((1, 1, CAP, D), lambda e, g: (g, e, 0, 0)),
        out_shape=jax.ShapeDtypeStruct((G, E, CAP, D), jnp.float32),
    )(a0, a1, expert_keys, expert_values)


def _ffn_half_b(a0, a1, expert_keys, expert_values, prev):
    eh = E // 2
    return pl.pallas_call(
        _ffn_b_body,
        grid=(E // 2, G),
        in_specs=[
            pl.BlockSpec((1, 1, CAP, D), lambda e, g: (g, e, 0, 0)),
            pl.BlockSpec((1, 1, CAP, D), lambda e, g: (g, e, 0, 0)),
            pl.BlockSpec((1, D, H), lambda e, g: (e + eh, 0, 0)),
            pl.BlockSpec((1, H, D), lambda e, g: (e + eh, 0, 0)),
            pl.BlockSpec(memory_space=pl.ANY),
        ],
        out_specs=pl.BlockSpec((1, 1, CAP, D), lambda e, g: (g, e + eh, 0, 0)),
        out_shape=jax.ShapeDtypeStruct((G, E, CAP, D), jnp.float32),
        input_output_aliases={4: 0},
    )(a0, a1, expert_keys, expert_values, prev)


_TOK_PER_W = G * S // NWORKERS
_CCHUNK = 16


def _cgather_body(aout, destc_hbm, r0, r1,
                  d0v, d1v, rows0, rows1, sem0, sem1):
    wid = lax.axis_index("s") * 2 + lax.axis_index("c")
    g = wid // (NWORKERS // G)
    tbase = (wid % (NWORKERS // G)) * _TOK_PER_W

    pltpu.sync_copy(destc_hbm.at[g, 0, pl.ds(tbase, _TOK_PER_W)], d0v)
    pltpu.sync_copy(destc_hbm.at[g, 1, pl.ds(tbase, _TOK_PER_W)], d1v)

    def body(j, _):
        off = j * _CCHUNK
        i0 = d0v[pl.ds(off, _CCHUNK)]
        i1 = d1v[pl.ds(off, _CCHUNK)]
        cp0 = pltpu.async_copy(aout.at[i0], rows0, sem0)
        cp1 = pltpu.async_copy(aout.at[i1], rows1, sem1)
        cp0.wait()
        cp1.wait()
        trow = g * S + tbase + off
        pltpu.sync_copy(rows0, r0.at[pl.ds(trow, _CCHUNK)])
        pltpu.sync_copy(rows1, r1.at[pl.ds(trow, _CCHUNK)])
        return 0
    lax.fori_loop(0, _TOK_PER_W // _CCHUNK, body, 0)


def _cgather(aout_flat, destc):
    mesh = plsc.VectorSubcoreMesh(core_axis_name="c", subcore_axis_name="s")
    f = functools.partial(
        pl.kernel,
        out_type=[jax.ShapeDtypeStruct((G * S, D), jnp.float32),
                  jax.ShapeDtypeStruct((G * S, D), jnp.float32)],
        mesh=mesh,
        scratch_types=[
            pltpu.VMEM((_TOK_PER_W,), jnp.int32),
            pltpu.VMEM((_TOK_PER_W,), jnp.int32),
            pltpu.VMEM((_CCHUNK, D), jnp.float32),
            pltpu.VMEM((_CCHUNK, D), jnp.float32),
            pltpu.SemaphoreType.DMA,
            pltpu.SemaphoreType.DMA,
        ],
    )(_cgather_body)
    return f(aout_flat, destc)


_SB = 512


def _cscale_body(r0_ref, r1_ref, g_ref, o_ref):
    gv = g_ref[0]
    o_ref[0] = gv[:, 0:1] * r0_ref[0] + gv[:, 1:2] * r1_ref[0]


def _cscale(r0, r1, gates_c):
    return pl.pallas_call(
        _cscale_body,
        grid=(G, S // _SB),
        in_specs=[
            pl.BlockSpec((1, _SB, D), lambda g, s: (g, s, 0)),
            pl.BlockSpec((1, _SB, D), lambda g, s: (g, s, 0)),
            pl.BlockSpec((1, _SB, 2), lambda g, s: (g, s, 0)),
        ],
        out_specs=pl.BlockSpec((1, _SB, D), lambda g, s: (g, s, 0)),
        out_shape=jax.ShapeDtypeStruct((G, S, D), jnp.float32),
    )(r0, r1, gates_c)


def kernel(x, gate_weight, gate_bias, expert_keys, expert_values):
    gates, destc, inv5, xpad3 = _router(x, gate_weight, gate_bias.reshape(E, 1))
    xpad = xpad3.reshape(G * XPAD, D)
    inv = inv5.reshape(G, 2, NSLOT)
    eh = E // 2
    a0A, a1A = _dispatch_half(xpad, inv, 0)
    a0B, a1B = _dispatch_half(xpad, inv, HNS)
    aoutA = _ffn_half_a(a0A.reshape(G, eh, CAP, D), a1A.reshape(G, eh, CAP, D),
                        expert_keys, expert_values)
    aout = _ffn_half_b(a0B.reshape(G, eh, CAP, D), a1B.reshape(G, eh, CAP, D),
                       expert_keys, expert_values, aoutA)
    r0, r1 = _cgather(aout.reshape(G * NSLOT, D), destc)
    out = _cscale(r0.reshape(G, S, D), r1.reshape(G, S, D),
                  jnp.transpose(gates, (0, 2, 1)))
    return out


# speedup vs baseline: 1.1078x; 1.0122x over previous
import functools

import jax
import jax.numpy as jnp
from jax import lax
from jax.experimental import pallas as pl
from jax.experimental.pallas import tpu as pltpu
from jax.experimental.pallas import tpu_sc as plsc

G = 2
S = 2048
D = 1024
H = 2048
E = 8
CAP = 512
NSLOT = E * CAP
ZROWS = 256
XPAD = S + ZROWS
NWORKERS = 32
LANES = 16
SLOT_BLK = 512


def _router_body(x_ref, w_ref, b_ref, gates_ref, destc_ref, inv_ref,
                 xpad_ref):
    g = pl.program_id(0)
    xg = x_ref[0]
    xpad_ref[0, :S] = xg
    xpad_ref[0, S:] = jnp.zeros((XPAD - S, D), jnp.float32)
    w = w_ref[...]
    b = b_ref[...]
    lt = lax.dot_general(w, xg, (((0,), (1,)), ((), ())),
                         preferred_element_type=jnp.float32)
    lt = lt + b

    eidx = lax.broadcasted_iota(jnp.int32, (E, S), 0)
    m1 = jnp.max(lt, axis=0, keepdims=True)
    am1 = jnp.min(jnp.where(lt == m1, eidx, E), axis=0, keepdims=True)
    lt2 = jnp.where(eidx == am1, jnp.float32(-1e30), lt)
    m2 = jnp.max(lt2, axis=0, keepdims=True)
    am2 = jnp.min(jnp.where(lt2 == m2, eidx, E), axis=0, keepdims=True)

    ssum = jnp.sum(jnp.exp(lt - m1), axis=0, keepdims=True)
    gate1 = 1.0 / ssum
    gate2 = jnp.exp(m2 - m1) / ssum

    oh1 = (eidx == am1).astype(jnp.float32)
    oh2 = (eidx == am2).astype(jnp.float32)
    m16 = jnp.concatenate([oh1, oh2], axis=0)

    ch = 512
    ut = (lax.broadcasted_iota(jnp.int32, (ch, ch), 0)
          <= lax.broadcasted_iota(jnp.int32, (ch, ch), 1)).astype(jnp.float32)
    prefix = jnp.zeros((16, 1), jnp.float32)
    chunks = []
    for i in range(S // ch):
        seg = m16[:, i * ch:(i + 1) * ch]
        cs = lax.dot_general(seg, ut, (((1,), (0,)), ((), ())),
                             preferred_element_type=jnp.float32) + prefix
        chunks.append(cs)
        prefix = cs[:, ch - 1:ch]
    csum = jnp.concatenate(chunks, axis=1)

    pos1 = jnp.sum(oh1 * csum[0:E], axis=0, keepdims=True)
    pos2 = jnp.sum(oh2 * csum[E:2 * E], axis=0, keepdims=True)

    valid1 = pos1 < jnp.float32(CAP)
    valid2 = pos2 < jnp.float32(CAP)
    gv1 = gate1 * valid1.astype(jnp.float32)
    gv2 = gate2 * valid2.astype(jnp.float32)
    d1 = jnp.where(valid1, am1 * CAP + pos1.astype(jnp.int32) - 1, NSLOT)
    d2 = jnp.where(valid2, am2 * CAP + pos2.astype(jnp.int32) - 1, NSLOT)

    gates_ref[0] = jnp.concatenate([gv1, gv2], axis=0)
    destc_ref[0] = jnp.concatenate(
        [jnp.minimum(d1, NSLOT - 1), jnp.minimum(d2, NSLOT - 1)],
        axis=0) + g * NSLOT

    zrow = g * XPAD + S
    svrow = (lax.broadcasted_iota(jnp.int32, (1, S), 1)
             + (g * XPAD - zrow))
    for k, drow in ((0, d1), (1, d2)):
        for blk in range(NSLOT // SLOT_BLK):
            siota = (lax.broadcasted_iota(jnp.int32, (SLOT_BLK, 1), 0)
                     + blk * SLOT_BLK)
            mb = (drow == siota)
            invcol = jnp.sum(jnp.where(mb, svrow, 0), axis=1, keepdims=True)
            spread = jnp.where(invcol == 0, siota % ZROWS, 0)
            inv_ref[0, k, blk] = invcol + spread + zrow


def _router(x, gate_weight, gate_bias2):
    nblk = NSLOT // SLOT_BLK
    return pl.pallas_call(
        _router_body,
        grid=(G,),
        in_specs=[
            pl.BlockSpec((1, S, D), lambda g: (g, 0, 0)),
            pl.BlockSpec((D, E), lambda g: (0, 0)),
            pl.BlockSpec((E, 1), lambda g: (0, 0)),
        ],
        out_specs=[
            pl.BlockSpec((1, 2, S), lambda g: (g, 0, 0)),
            pl.BlockSpec((1, 2, S), lambda g: (g, 0, 0)),
            pl.BlockSpec((1, 2, nblk, SLOT_BLK, 1), lambda g: (g, 0, 0, 0, 0)),
            pl.BlockSpec((1, XPAD, D), lambda g: (g, 0, 0)),
        ],
        out_shape=[
            jax.ShapeDtypeStruct((G, 2, S), jnp.float32),
            jax.ShapeDtypeStruct((G, 2, S), jnp.int32),
            jax.ShapeDtypeStruct((G, 2, nblk, SLOT_BLK, 1), jnp.int32),
            jax.ShapeDtypeStruct((G, XPAD, D), jnp.float32),
        ],
    )(x, gate_weight, gate_bias2)


HNS = NSLOT // 2
_SLOTS_PER_W = G * HNS // NWORKERS
_DCHUNK = 32


def _dispatch_half_body(soff, xpad, inv_hbm, a0, a1,
                        idx0v, idx1v, rows0, rows1, sem0, sem1):
    wid = lax.axis_index("s") * 2 + lax.axis_index("c")
    g = wid // (NWORKERS // G)
    lbase = (wid % (NWORKERS // G)) * _SLOTS_PER_W

    pltpu.sync_copy(inv_hbm.at[g, 0, pl.ds(soff + lbase, _SLOTS_PER_W)], idx0v)
    pltpu.sync_copy(inv_hbm.at[g, 1, pl.ds(soff + lbase, _SLOTS_PER_W)], idx1v)

    def gbody(j, _):
        off = j * _DCHUNK
        cp0 = pltpu.async_copy(
            xpad.at[idx0v.at[pl.ds(off, _DCHUNK)]], rows0, sem0)
        cp1 = pltpu.async_copy(
            xpad.at[idx1v.at[pl.ds(off, _DCHUNK)]], rows1, sem1)
        cp0.wait()
        cp1.wait()
        grow = g * HNS + lbase + off
        pltpu.sync_copy(rows0, a0.at[pl.ds(grow, _DCHUNK)])
        pltpu.sync_copy(rows1, a1.at[pl.ds(grow, _DCHUNK)])
        return 0
    lax.fori_loop(0, _SLOTS_PER_W // _DCHUNK, gbody, 0)


def _dispatch_half(xpad, inv, soff):
    mesh = plsc.VectorSubcoreMesh(core_axis_name="c", subcore_axis_name="s")
    f = functools.partial(
        pl.kernel,
        out_type=[jax.ShapeDtypeStruct((G * HNS, D), jnp.float32),
                  jax.ShapeDtypeStruct((G * HNS, D), jnp.float32)],
        mesh=mesh,
        scratch_types=[
            pltpu.VMEM((_SLOTS_PER_W,), jnp.int32),
            pltpu.VMEM((_SLOTS_PER_W,), jnp.int32),
            pltpu.VMEM((_DCHUNK, D), jnp.float32),
            pltpu.VMEM((_DCHUNK, D), jnp.float32),
            pltpu.SemaphoreType.DMA,
            pltpu.SemaphoreType.DMA,
        ],
    )(functools.partial(_dispatch_half_body, soff))
    return f(xpad, inv)


def _ffn_body(a0_ref, a1_ref, k_ref, v_ref, o_ref):
    a = (a0_ref[0, 0] + a1_ref[0, 0]).astype(jnp.bfloat16)
    h = jnp.dot(a, k_ref[0].astype(jnp.bfloat16),
                preferred_element_type=jnp.float32)
    h = jax.nn.gelu(h).astype(jnp.bfloat16)
    o_ref[0, 0] = jnp.dot(h, v_ref[0].astype(jnp.bfloat16),
                          preferred_element_type=jnp.float32)


def _ffn_b_body(a0_ref, a1_ref, k_ref, v_ref, prev_ref, o_ref):
    del prev_ref
    _ffn_body(a0_ref, a1_ref, k_ref, v_ref, o_ref)


def _ffn_half_a(a0, a1, expert_keys, expert_values):
    return pl.pallas_call(
        _ffn_body,
        grid=(E // 2, G),
        in_specs=[
            pl.BlockSpec((1, 1, CAP, D), lambda e, g: (g, e, 0, 0)),
            pl.BlockSpec((1, 1, CAP, D), lambda e, g: (g, e, 0, 0)),
            pl.BlockSpec((1, D, H), lambda e, g: (e, 0, 0)),
            pl.BlockSpec((1, H, D), lambda e, g: (e, 0, 0)),
        ],
        out_specs=pl.BlockSpec((1, 1, CAP, D), lambda e, g: (g, e, 0, 0)),
        out_shape=jax.ShapeDtypeStruct((G, E, CAP, D), jnp.float32),
    )(a0, a1, expert_keys, expert_values)


def _ffn_half_b(a0, a1, expert_keys, expert_values, prev):
    eh = E // 2
    return pl.pallas_call(
        _ffn_b_body,
        grid=(E // 2, G),
        in_specs=[
            pl.BlockSpec((1, 1, CAP, D), lambda e, g: (g, e, 0, 0)),
            pl.BlockSpec((1, 1, CAP, D), lambda e, g: (g, e, 0, 0)),
            pl.BlockSpec((1, D, H), lambda e, g: (e + eh, 0, 0)),
            pl.BlockSpec((1, H, D), lambda e, g: (e + eh, 0, 0)),
            pl.BlockSpec(memory_space=pl.ANY),
        ],
        out_specs=pl.BlockSpec((1, 1, CAP, D), lambda e, g: (g, e + eh, 0, 0)),
        out_shape=jax.ShapeDtypeStruct((G, E, CAP, D), jnp.float32),
        input_output_aliases={4: 0},
    )(a0, a1, expert_keys, expert_values, prev)


_TOK_PER_W = G * S // NWORKERS
_CCHUNK = 16


def _cgather_body(aout, destc_hbm, r0, r1,
                  d0v, d1v, rows0, rows1, sem0, sem1):
    wid = lax.axis_index("s") * 2 + lax.axis_index("c")
    g = wid // (NWORKERS // G)
    tbase = (wid % (NWORKERS // G)) * _TOK_PER_W

    pltpu.sync_copy(destc_hbm.at[g, 0, pl.ds(tbase, _TOK_PER_W)], d0v)
    pltpu.sync_copy(destc_hbm.at[g, 1, pl.ds(tbase, _TOK_PER_W)], d1v)

    def body(j, _):
        off = j * _CCHUNK
        i0 = d0v[pl.ds(off, _CCHUNK)]
        i1 = d1v[pl.ds(off, _CCHUNK)]
        cp0 = pltpu.async_copy(aout.at[i0], rows0, sem0)
        cp1 = pltpu.async_copy(aout.at[i1], rows1, sem1)
        cp0.wait()
        cp1.wait()
        trow = g * S + tbase + off
        pltpu.sync_copy(rows0, r0.at[pl.ds(trow, _CCHUNK)])
        pltpu.sync_copy(rows1, r1.at[pl.ds(trow, _CCHUNK)])
        return 0
    lax.fori_loop(0, _TOK_PER_W // _CCHUNK, body, 0)


def _cgather(aout_flat, destc):
    mesh = plsc.VectorSubcoreMesh(core_axis_name="c", subcore_axis_name="s")
    f = functools.partial(
        pl.kernel,
        out_type=[jax.ShapeDtypeStruct((G * S, D), jnp.float32),
                  jax.ShapeDtypeStruct((G * S, D), jnp.float32)],
        mesh=mesh,
        scratch_types=[
            pltpu.VMEM((_TOK_PER_W,), jnp.int32),
            pltpu.VMEM((_TOK_PER_W,), jnp.int32),
            pltpu.VMEM((_CCHUNK, D), jnp.float32),
            pltpu.VMEM((_CCHUNK, D), jnp.float32),
            pltpu.SemaphoreType.DMA,
            pltpu.SemaphoreType.DMA,
        ],
    )(_cgather_body)
    return f(aout_flat, destc)


_SB = 512


def _cscale_body(r0_ref, r1_ref, g_ref, o_ref):
    gv = g_ref[0]
    o_ref[0] = gv[:, 0:1] * r0_ref[0] + gv[:, 1:2] * r1_ref[0]


def _cscale(r0, r1, gates_c):
    return pl.pallas_call(
        _cscale_body,
        grid=(G, S // _SB),
        in_specs=[
            pl.BlockSpec((1, _SB, D), lambda g, s: (g, s, 0)),
            pl.BlockSpec((1, _SB, D), lambda g, s: (g, s, 0)),
            pl.BlockSpec((1, _SB, 2), lambda g, s: (g, s, 0)),
        ],
        out_specs=pl.BlockSpec((1, _SB, D), lambda g, s: (g, s, 0)),
        out_shape=jax.ShapeDtypeStruct((G, S, D), jnp.float32),
    )(r0, r1, gates_c)


def kernel(x, gate_weight, gate_bias, expert_keys, expert_values):
    gates, destc, inv5, xpad3 = _router(x, gate_weight, gate_bias.reshape(E, 1))
    xpad = xpad3.reshape(G * XPAD, D)
    inv = inv5.reshape(G, 2, NSLOT)
    eh = E // 2
    a0A, a1A = _dispatch_half(xpad, inv, 0)
    a0B, a1B = _dispatch_half(xpad, inv, HNS)
    aoutA = _ffn_half_a(a0A.reshape(G, eh, CAP, D), a1A.reshape(G, eh, CAP, D),
                        expert_keys, expert_values)
    aout = _ffn_half_b(a0B.reshape(G, eh, CAP, D), a1B.reshape(G, eh, CAP, D),
                       expert_keys, expert_values, aoutA)
    r0, r1 = _cgather(aout.reshape(G * NSLOT, D), destc)
    out = _cscale(r0.reshape(G, S, D), r1.reshape(G, S, D),
                  jnp.transpose(gates, (0, 2, 1)))
    return out


# speedup vs baseline: 1.1158x; 1.0073x over previous
import functools

import jax
import jax.numpy as jnp
from jax import lax
from jax.experimental import pallas as pl
from jax.experimental.pallas import tpu as pltpu
from jax.experimental.pallas import tpu_sc as plsc

G = 2
S = 2048
D = 1024
H = 2048
E = 8
CAP = 512
NSLOT = E * CAP
ZROWS = 256
XPAD = S + ZROWS
NWORKERS = 32
LANES = 16
SLOT_BLK = 512


def _router_body(x_ref, w_ref, b_ref, gates_ref, destc_ref, inv_ref,
                 xpad_ref):
    g = pl.program_id(0)
    xg = x_ref[0]
    xpad_ref[0, :S] = xg
    xpad_ref[0, S:] = jnp.zeros((XPAD - S, D), jnp.float32)
    w = w_ref[...]
    b = b_ref[...]
    lt = lax.dot_general(w, xg, (((0,), (1,)), ((), ())),
                         preferred_element_type=jnp.float32)
    lt = lt + b

    eidx = lax.broadcasted_iota(jnp.int32, (E, S), 0)
    m1 = jnp.max(lt, axis=0, keepdims=True)
    am1 = jnp.min(jnp.where(lt == m1, eidx, E), axis=0, keepdims=True)
    lt2 = jnp.where(eidx == am1, jnp.float32(-1e30), lt)
    m2 = jnp.max(lt2, axis=0, keepdims=True)
    am2 = jnp.min(jnp.where(lt2 == m2, eidx, E), axis=0, keepdims=True)

    ssum = jnp.sum(jnp.exp(lt - m1), axis=0, keepdims=True)
    gate1 = 1.0 / ssum
    gate2 = jnp.exp(m2 - m1) / ssum

    oh1 = (eidx == am1).astype(jnp.float32)
    oh2 = (eidx == am2).astype(jnp.float32)
    m16 = jnp.concatenate([oh1, oh2], axis=0)

    ch = 512
    ut = (lax.broadcasted_iota(jnp.int32, (ch, ch), 0)
          <= lax.broadcasted_iota(jnp.int32, (ch, ch), 1)).astype(jnp.float32)
    prefix = jnp.zeros((16, 1), jnp.float32)
    chunks = []
    for i in range(S // ch):
        seg = m16[:, i * ch:(i + 1) * ch]
        cs = lax.dot_general(seg, ut, (((1,), (0,)), ((), ())),
                             preferred_element_type=jnp.float32) + prefix
        chunks.append(cs)
        prefix = cs[:, ch - 1:ch]
    csum = jnp.concatenate(chunks, axis=1)

    pos1 = jnp.sum(oh1 * csum[0:E], axis=0, keepdims=True)
    pos2 = jnp.sum(oh2 * csum[E:2 * E], axis=0, keepdims=True)

    valid1 = pos1 < jnp.float32(CAP)
    valid2 = pos2 < jnp.float32(CAP)
    gv1 = gate1 * valid1.astype(jnp.float32)
    gv2 = gate2 * valid2.astype(jnp.float32)
    d1 = jnp.where(valid1, am1 * CAP + pos1.astype(jnp.int32) - 1, NSLOT)
    d2 = jnp.where(valid2, am2 * CAP + pos2.astype(jnp.int32) - 1, NSLOT)

    gates_ref[0] = jnp.concatenate([gv1, gv2], axis=0)
    destc_ref[0] = jnp.concatenate(
        [jnp.minimum(d1, NSLOT - 1), jnp.minimum(d2, NSLOT - 1)],
        axis=0) + g * NSLOT

    zrow = g * XPAD + S
    svrow = (lax.broadcasted_iota(jnp.int32, (1, S), 1)
             + (g * XPAD - zrow))
    for k, drow in ((0, d1), (1, d2)):
        for blk in range(NSLOT // SLOT_BLK):
            siota = (lax.broadcasted_iota(jnp.int32, (SLOT_BLK, 1), 0)
                     + blk * SLOT_BLK)
            mb = (drow == siota)
            invcol = jnp.sum(jnp.where(mb, svrow, 0), axis=1, keepdims=True)
            spread = jnp.where(invcol == 0, siota % ZROWS, 0)
            inv_ref[0, k, blk] = invcol + spread + zrow


def _router(x, gate_weight, gate_bias2):
    nblk = NSLOT // SLOT_BLK
    return pl.pallas_call(
        _router_body,
        grid=(G,),
        in_specs=[
            pl.BlockSpec((1, S, D), lambda g: (g, 0, 0)),
            pl.BlockSpec((D, E), lambda g: (0, 0)),
            pl.BlockSpec((E, 1), lambda g: (0, 0)),
        ],
        out_specs=[
            pl.BlockSpec((1, 2, S), lambda g: (g, 0, 0)),
            pl.BlockSpec((1, 2, S), lambda g: (g, 0, 0)),
            pl.BlockSpec((1, 2, nblk, SLOT_BLK, 1), lambda g: (g, 0, 0, 0, 0)),
            pl.BlockSpec((1, XPAD, D), lambda g: (g, 0, 0)),
        ],
        out_shape=[
            jax.ShapeDtypeStruct((G, 2, S), jnp.float32),
            jax.ShapeDtypeStruct((G, 2, S), jnp.int32),
            jax.ShapeDtypeStruct((G, 2, nblk, SLOT_BLK, 1), jnp.int32),
            jax.ShapeDtypeStruct((G, XPAD, D), jnp.float32),
        ],
    )(x, gate_weight, gate_bias2)


HNS = NSLOT // 2
_SLOTS_PER_W = G * HNS // NWORKERS
_DCHUNK = 32


def _dispatch_half_body(soff, xpad, inv_hbm, a0, a1,
                        idx0v, idx1v, rows0, rows1, sem0, sem1):
    wid = lax.axis_index("s") * 2 + lax.axis_index("c")
    g = wid // (NWORKERS // G)
    lbase = (wid % (NWORKERS // G)) * _SLOTS_PER_W

    pltpu.sync_copy(inv_hbm.at[g, 0, pl.ds(soff + lbase, _SLOTS_PER_W)], idx0v)
    pltpu.sync_copy(inv_hbm.at[g, 1, pl.ds(soff + lbase, _SLOTS_PER_W)], idx1v)

    def gbody(j, _):
        off = j * _DCHUNK
        cp0 = pltpu.async_copy(
            xpad.at[idx0v.at[pl.ds(off, _DCHUNK)]], rows0, sem0)
        cp1 = pltpu.async_copy(
            xpad.at[idx1v.at[pl.ds(off, _DCHUNK)]], rows1, sem1)
        cp0.wait()
        cp1.wait()
        grow = g * HNS + lbase + off
        pltpu.sync_copy(rows0, a0.at[pl.ds(grow, _DCHUNK)])
        pltpu.sync_copy(rows1, a1.at[pl.ds(grow, _DCHUNK)])
        return 0
    lax.fori_loop(0, _SLOTS_PER_W // _DCHUNK, gbody, 0)


def _dispatch_half(xpad, inv, soff):
    mesh = plsc.VectorSubcoreMesh(core_axis_name="c", subcore_axis_name="s")
    f = functools.partial(
        pl.kernel,
        out_type=[jax.ShapeDtypeStruct((G * HNS, D), jnp.float32),
                  jax.ShapeDtypeStruct((G * HNS, D), jnp.float32)],
        mesh=mesh,
        scratch_types=[
            pltpu.VMEM((_SLOTS_PER_W,), jnp.int32),
            pltpu.VMEM((_SLOTS_PER_W,), jnp.int32),
            pltpu.VMEM((_DCHUNK, D), jnp.float32),
            pltpu.VMEM((_DCHUNK, D), jnp.float32),
            pltpu.SemaphoreType.DMA,
            pltpu.SemaphoreType.DMA,
        ],
    )(functools.partial(_dispatch_half_body, soff))
    return f(xpad, inv)


def _ffn_body(a0_ref, a1_ref, k_ref, v_ref, o_ref):
    a = (a0_ref[0, 0] + a1_ref[0, 0]).astype(jnp.bfloat16)
    h = jnp.dot(a, k_ref[0].astype(jnp.bfloat16),
                preferred_element_type=jnp.float32)
    h = jax.nn.gelu(h).astype(jnp.bfloat16)
    o_ref[0, 0] = jnp.dot(h, v_ref[0].astype(jnp.bfloat16),
                          preferred_element_type=jnp.float32)


def _ffn_b_body(a0_ref, a1_ref, k_ref, v_ref, prev_ref, o_ref):
    del prev_ref
    _ffn_body(a0_ref, a1_ref, k_ref, v_ref, o_ref)


def _ffn_half_a(a0, a1, expert_keys, expert_values):
    return pl.pallas_call(
        _ffn_body,
        grid=(E // 2, G),
        in_specs=[
            pl.BlockSpec((1, 1, CAP, D), lambda e, g: (g, e, 0, 0)),
            pl.BlockSpec((1, 1, CAP, D), lambda e, g: (g, e, 0, 0)),
            pl.BlockSpec((1, D, H), lambda e, g: (e, 0, 0)),
            pl.BlockSpec((1, H, D), lambda e, g: (e, 0, 0)),
        ],
        out_specs=pl.BlockSpec((1, 1, CAP, D), lambda e, g: (g, e, 0, 0)),
        out_shape=jax.ShapeDtypeStruct((G, E, CAP, D), jnp.float32),
    )(a0, a1, expert_keys, expert_values)


def _ffn_half_b(a0, a1, expert_keys, expert_values, prev):
    eh = E // 2
    return pl.pallas_call(
        _ffn_b_body,
        grid=(E // 2, G),
        in_specs=[
            pl.BlockSpec((1, 1, CAP, D), lambda e, g: (g, e, 0, 0)),
            pl.BlockSpec((1, 1, CAP, D), lambda e, g: (g, e, 0, 0)),
            pl.BlockSpec((1, D, H), lambda e, g: (e + eh, 0, 0)),
            pl.BlockSpec((1, H, D), lambda e, g: (e + eh, 0, 0)),
            pl.BlockSpec(memory_space=pl.ANY),
        ],
        out_specs=pl.BlockSpec((1, 1, CAP, D), lambda e, g: (g, e + eh, 0, 0)),
        out_shape=jax.ShapeDtypeStruct((G, E, CAP, D), jnp.float32),
        input_output_aliases={4: 0},
    )(a0, a1, expert_keys, expert_values, prev)


_TOK_PER_W = G * S // NWORKERS
_CCHUNK = 32


def _cgather_body(aout, destc_hbm, r0, r1,
                  d0v, d1v, rows0, rows1, sem0, sem1):
    wid = lax.axis_index("s") * 2 + lax.axis_index("c")
    g = wid // (NWORKERS // G)
    tbase = (wid % (NWORKERS // G)) * _TOK_PER_W

    pltpu.sync_copy(destc_hbm.at[g, 0, pl.ds(tbase, _TOK_PER_W)], d0v)
    pltpu.sync_copy(destc_hbm.at[g, 1, pl.ds(tbase, _TOK_PER_W)], d1v)

    def body(j, _):
        off = j * _CCHUNK
        cp0 = pltpu.async_copy(
            aout.at[d0v.at[pl.ds(off, _CCHUNK)]], rows0, sem0)
        cp1 = pltpu.async_copy(
            aout.at[d1v.at[pl.ds(off, _CCHUNK)]], rows1, sem1)
        cp0.wait()
        cp1.wait()
        trow = g * S + tbase + off
        pltpu.sync_copy(rows0, r0.at[pl.ds(trow, _CCHUNK)])
        pltpu.sync_copy(rows1, r1.at[pl.ds(trow, _CCHUNK)])
        return 0
    lax.fori_loop(0, _TOK_PER_W // _CCHUNK, body, 0)


def _cgather(aout_flat, destc):
    mesh = plsc.VectorSubcoreMesh(core_axis_name="c", subcore_axis_name="s")
    f = functools.partial(
        pl.kernel,
        out_type=[jax.ShapeDtypeStruct((G * S, D), jnp.float32),
                  jax.ShapeDtypeStruct((G * S, D), jnp.float32)],
        mesh=mesh,
        scratch_types=[
            pltpu.VMEM((_TOK_PER_W,), jnp.int32),
            pltpu.VMEM((_TOK_PER_W,), jnp.int32),
            pltpu.VMEM((_CCHUNK, D), jnp.float32),
            pltpu.VMEM((_CCHUNK, D), jnp.float32),
            pltpu.SemaphoreType.DMA,
            pltpu.SemaphoreType.DMA,
        ],
    )(_cgather_body)
    return f(aout_flat, destc)


_SB = 512


def _cscale_body(r0_ref, r1_ref, g_ref, o_ref):
    gv = g_ref[0]
    o_ref[0] = gv[:, 0:1] * r0_ref[0] + gv[:, 1:2] * r1_ref[0]


def _cscale(r0, r1, gates_c):
    return pl.pallas_call(
        _cscale_body,
        grid=(G, S // _SB),
        in_specs=[
            pl.BlockSpec((1, _SB, D), lambda g, s: (g, s, 0)),
            pl.BlockSpec((1, _SB, D), lambda g, s: (g, s, 0)),
            pl.BlockSpec((1, _SB, 2), lambda g, s: (g, s, 0)),
        ],
        out_specs=pl.BlockSpec((1, _SB, D), lambda g, s: (g, s, 0)),
        out_shape=jax.ShapeDtypeStruct((G, S, D), jnp.float32),
    )(r0, r1, gates_c)


def kernel(x, gate_weight, gate_bias, expert_keys, expert_values):
    gates, destc, inv5, xpad3 = _router(x, gate_weight, gate_bias.reshape(E, 1))
    xpad = xpad3.reshape(G * XPAD, D)
    inv = inv5.reshape(G, 2, NSLOT)
    eh = E // 2
    a0A, a1A = _dispatch_half(xpad, inv, 0)
    a0B, a1B = _dispatch_half(xpad, inv, HNS)
    aoutA = _ffn_half_a(a0A.reshape(G, eh, CAP, D), a1A.reshape(G, eh, CAP, D),
                        expert_keys, expert_values)
    aout = _ffn_half_b(a0B.reshape(G, eh, CAP, D), a1B.reshape(G, eh, CAP, D),
                       expert_keys, expert_values, aoutA)
    r0, r1 = _cgather(aout.reshape(G * NSLOT, D), destc)
    out = _cscale(r0.reshape(G, S, D), r1.reshape(G, S, D),
                  jnp.transpose(gates, (0, 2, 1)))
    return out
